# baseline (speedup 1.0000x reference)
import functools

import jax
import jax.numpy as jnp
from jax import lax
from jax.experimental import pallas as pl
from jax.experimental.pallas import tpu as pltpu
from jax.experimental.pallas import tpu_sc as plsc

NC, NS = 2, 16
NW = NC * NS
CHUNK = 128


@functools.partial(jax.jit, static_argnames=("total", "d"))
def _sc_gather(idx2d, table, total, d):
    per_w = total // NW
    n_chunks = per_w // CHUNK
    mesh = plsc.VectorSubcoreMesh(core_axis_name="c", subcore_axis_name="s")

    @functools.partial(
        pl.kernel,
        mesh=mesh,
        out_type=jax.ShapeDtypeStruct((total, d), jnp.float32),
        scratch_types=[
            pltpu.VMEM((n_chunks, CHUNK), jnp.int32),
            pltpu.VMEM((CHUNK, d), jnp.float32),
            pltpu.SemaphoreType.DMA,
        ],
        compiler_params=pltpu.CompilerParams(use_tc_tiling_on_sc=False),
    )
    def k(idx_hbm, table_hbm, out_hbm, idx_v, rows_v, sem):
        wid = lax.axis_index("s") * NC + lax.axis_index("c")
        base = wid * per_w
        pltpu.sync_copy(idx_hbm.at[pl.ds(wid * n_chunks, n_chunks)], idx_v)

        def body(c, _):
            pltpu.async_copy(table_hbm.at[idx_v.at[c]], rows_v, sem).wait()
            pltpu.sync_copy(rows_v, out_hbm.at[pl.ds(base + c * CHUNK, CHUNK)])
            return _

        lax.fori_loop(0, n_chunks, body, None)

    return k(idx2d, table)


def kernel(X, table):
    b, s = X.shape
    total = b * s
    d = table.shape[1]
    idx2d = X.astype(jnp.int32).reshape(total // CHUNK, CHUNK)
    out = _sc_gather(idx2d, table, total, d)
    return out.reshape(b, s, d)


# speedup vs baseline: 1.1125x; 1.1125x over previous
import functools

import jax
import jax.numpy as jnp
from jax import lax
from jax.experimental import pallas as pl
from jax.experimental.pallas import tpu as pltpu
from jax.experimental.pallas import tpu_sc as plsc

NC, NS = 2, 16
NW = NC * NS
CHUNK = 128
BLK = 4


@functools.partial(jax.jit, static_argnames=("total", "d"))
def _sc_gather(idx2d, table, total, d):
    per_w = total // NW
    n_chunks = per_w // CHUNK
    n_blocks = n_chunks // BLK
    mesh = plsc.VectorSubcoreMesh(core_axis_name="c", subcore_axis_name="s")

    @functools.partial(
        pl.kernel,
        mesh=mesh,
        out_type=jax.ShapeDtypeStruct((total, d), jnp.float32),
        scratch_types=[
            pltpu.VMEM((n_chunks, CHUNK), jnp.int32),
            pltpu.VMEM((2 * BLK, CHUNK, d), jnp.float32),
            pltpu.SemaphoreType.DMA,
            pltpu.SemaphoreType.DMA,
        ],
        compiler_params=pltpu.CompilerParams(use_tc_tiling_on_sc=False),
    )
    def k(idx_hbm, table_hbm, out_hbm, idx_v, rows_v, gsem, osem):
        wid = lax.axis_index("s") * NC + lax.axis_index("c")
        base = wid * per_w
        pltpu.sync_copy(idx_hbm.at[pl.ds(wid * n_chunks, n_chunks)], idx_v)

        def gather(c, slot):
            pltpu.async_copy(table_hbm.at[idx_v.at[c]], rows_v.at[slot], gsem)

        def drain_gathers():
            for _ in range(BLK):
                pltpu.make_async_copy(
                    table_hbm.at[idx_v.at[0]], rows_v.at[0], gsem
                ).wait()

        def drain_outs():
            for _ in range(BLK):
                pltpu.make_async_copy(
                    rows_v.at[0], out_hbm.at[pl.ds(base, CHUNK)], osem
                ).wait()

        for b in range(BLK):
            gather(b, b)

        def body(g, carry):
            cur = (g % 2) * BLK

            @pl.when(g > 0)
            def _():
                drain_outs()

            drain_gathers()

            for b in range(BLK):
                c = g * BLK + b
                pltpu.async_copy(
                    rows_v.at[cur + b],
                    out_hbm.at[pl.ds(base + c * CHUNK, CHUNK)],
                    osem,
                )

            @pl.when(g + 1 < n_blocks)
            def _():
                for b in range(BLK):
                    gather((g + 1) * BLK + b, (BLK - cur) + b)

            return carry

        lax.fori_loop(0, n_blocks, body, None)
        drain_outs()

    return k(idx2d, table)


def kernel(X, table):
    b, s = X.shape
    total = b * s
    d = table.shape[1]
    idx2d = X.astype(jnp.int32).reshape(total // CHUNK, CHUNK)
    out = _sc_gather(idx2d, table, total, d)
    return out.reshape(b, s, d)


# speedup vs baseline: 1.1157x; 1.0029x over previous
import functools

import jax
import jax.numpy as jnp
from jax import lax
from jax.experimental import pallas as pl
from jax.experimental.pallas import tpu as pltpu
from jax.experimental.pallas import tpu_sc as plsc

NC, NS = 2, 16
NW = NC * NS
CHUNK = 128
BLK = 4


@functools.partial(jax.jit, static_argnames=("total", "d"))
def _sc_gather(idx2d, table, total, d):
    per_w = total // NW
    n_chunks = per_w // CHUNK
    n_blocks = n_chunks // BLK
    mesh = plsc.VectorSubcoreMesh(core_axis_name="c", subcore_axis_name="s")

    @functools.partial(
        pl.kernel,
        mesh=mesh,
        out_type=jax.ShapeDtypeStruct((total, d), jnp.float32),
        scratch_types=[
            pltpu.VMEM((n_chunks, CHUNK), jnp.int32),
            pltpu.VMEM((2 * BLK, CHUNK, d), jnp.float32),
            pltpu.SemaphoreType.DMA,
            pltpu.SemaphoreType.DMA,
        ],
        compiler_params=pltpu.CompilerParams(
            use_tc_tiling_on_sc=False,
            skip_device_barrier=True,
            disable_bounds_checks=True,
            disable_semaphore_checks=True,
        ),
    )
    def k(idx_hbm, table_hbm, out_hbm, idx_v, rows_v, gsem, osem):
        wid = lax.axis_index("s") * NC + lax.axis_index("c")
        base = wid * per_w
        pltpu.sync_copy(idx_hbm.at[pl.ds(wid * n_chunks, n_chunks)], idx_v)

        def gather(c, slot):
            pltpu.async_copy(table_hbm.at[idx_v.at[c]], rows_v.at[slot], gsem)

        def drain_gathers():
            for _ in range(BLK):
                pltpu.make_async_copy(
                    table_hbm.at[idx_v.at[0]], rows_v.at[0], gsem
                ).wait()

        def drain_outs():
            for _ in range(BLK):
                pltpu.make_async_copy(
                    rows_v.at[0], out_hbm.at[pl.ds(base, CHUNK)], osem
                ).wait()

        for b in range(BLK):
            gather(b, b)

        def body(g, carry):
            cur = (g % 2) * BLK

            @pl.when(g > 0)
            def _():
                drain_outs()

            drain_gathers()

            for b in range(BLK):
                c = g * BLK + b
                pltpu.async_copy(
                    rows_v.at[cur + b],
                    out_hbm.at[pl.ds(base + c * CHUNK, CHUNK)],
                    osem,
                )

            @pl.when(g + 1 < n_blocks)
            def _():
                for b in range(BLK):
                    gather((g + 1) * BLK + b, (BLK - cur) + b)

            return carry

        lax.fori_loop(0, n_blocks, body, None)
        drain_outs()

    return k(idx2d, table)


def kernel(X, table):
    b, s = X.shape
    total = b * s
    d = table.shape[1]
    idx2d = X.astype(jnp.int32).reshape(total // CHUNK, CHUNK)
    out = _sc_gather(idx2d, table, total, d)
    return out.reshape(b, s, d)
